# baseline (speedup 1.0000x reference)
import functools
import math

import jax
import jax.numpy as jnp
from jax import lax
from jax.experimental import pallas as pl
from jax.experimental.pallas import tpu as pltpu
from jax.experimental.pallas import tpu_sc as plsc

_N = 10000
_E = 320000
_D = 128
_N_LAYERS = 4
_ALPHA = 0.1
_LAMDA = 0.5

_NC = 2
_NS = 16
_NW = _NC * _NS

_CH = 64
_CPW = 168
_EPAD = _CH * _CPW * _NW
_NBUF = 3
_NPH = 2
_CPP = _CPW // _NPH
_SRCP = 96
_DSTP = 88
_SRC_RPW = _NPH * _SRCP
_DST_RPW = _NPH * _DSTP

_NPAD = 10240
_RPT = _NPAD // _NS

_R = 1024
_GRID = _NPAD // _R


def _sc_agg_body(g_hbm, src_hbm, dst_hbm, zeros_hbm, out_hbm,
                 src_v, dst_v, acc, *bufs_and_sems):
    bufs = bufs_and_sems[:_NBUF]
    gsem = bufs_and_sems[_NBUF:2 * _NBUF]
    ssem = bufs_and_sems[2 * _NBUF:3 * _NBUF]
    c = lax.axis_index("c")
    s = lax.axis_index("s")
    wid = s * _NC + c
    pltpu.sync_copy(zeros_hbm.at[pl.ds(s * _RPT, _RPT)],
                    acc.at[pl.ds(s * _RPT, _RPT)])
    plsc.subcore_barrier()

    for p in range(_NPH):
        pltpu.sync_copy(src_hbm.at[pl.ds(wid * _SRC_RPW + p * _SRCP, _SRCP)],
                        src_v)
        pltpu.sync_copy(dst_hbm.at[pl.ds(wid * _DST_RPW + p * _DSTP, _DSTP)],
                        dst_v)
        for b in range(_NBUF):
            pltpu.async_copy(g_hbm.at[src_v.at[b]], bufs[b], gsem[b])

        def group(i, carry):
            j = i * _NBUF
            for b in range(_NBUF):
                pltpu.make_async_copy(g_hbm.at[src_v.at[j + b]],
                                      bufs[b], gsem[b]).wait()
                pltpu.async_copy(bufs[b], acc.at[dst_v.at[j + b]], ssem[b],
                                 add=True)
            for b in range(_NBUF):
                pltpu.make_async_copy(bufs[b], acc.at[dst_v.at[j + b]],
                                      ssem[b]).wait()
                pltpu.async_copy(g_hbm.at[src_v.at[j + b + _NBUF]],
                                 bufs[b], gsem[b])
            return carry

        lax.fori_loop(0, _CPP // _NBUF, group, 0)
        for b in range(_NBUF):
            pltpu.make_async_copy(g_hbm.at[src_v.at[b]], bufs[b],
                                  gsem[b]).wait()
    plsc.subcore_barrier()
    pltpu.sync_copy(acc.at[pl.ds(s * _RPT, _RPT)],
                    out_hbm.at[pl.ds(c * _NPAD + s * _RPT, _RPT)])


@jax.jit
def _sc_agg(g, src3, dst3, zeros128):
    fn = pl.kernel(
        _sc_agg_body,
        out_type=jax.ShapeDtypeStruct((2 * _NPAD, _D), jnp.float32),
        mesh=plsc.VectorSubcoreMesh(core_axis_name="c", subcore_axis_name="s"),
        scratch_types=(
            [pltpu.VMEM((_SRCP, _CH), jnp.int32),
             pltpu.VMEM((_DSTP, _CH), jnp.int32),
             pltpu.VMEM_SHARED((_NPAD, _D), jnp.float32)]
            + [pltpu.VMEM((_CH, _D), jnp.float32) for _ in range(_NBUF)]
            + [pltpu.SemaphoreType.DMA for _ in range(2 * _NBUF)]
        ),
    )
    return fn(g, src3, dst3, zeros128)


def _sc_deg_body(dst_hbm, zeros_hbm, ones_hbm, out_hbm,
                 dst_v, ones_v, acc, sem):
    c = lax.axis_index("c")
    s = lax.axis_index("s")
    wid = s * _NC + c
    pltpu.sync_copy(zeros_hbm.at[pl.ds(s * _RPT, _RPT)],
                    acc.at[pl.ds(s * _RPT, _RPT)])
    pltpu.sync_copy(dst_hbm.at[pl.ds(wid * _DST_RPW, _DST_RPW)], dst_v)
    pltpu.sync_copy(ones_hbm, ones_v)
    plsc.subcore_barrier()

    def chunk(j, carry):
        pltpu.sync_copy(ones_v, acc.at[dst_v.at[j]], add=True)
        return carry

    lax.fori_loop(0, _DST_RPW, chunk, 0)
    plsc.subcore_barrier()
    pltpu.sync_copy(acc.at[pl.ds(s * _RPT, _RPT)],
                    out_hbm.at[pl.ds(c * _NPAD + s * _RPT, _RPT)])


@jax.jit
def _sc_degree(dst3, zeros128, ones128):
    fn = pl.kernel(
        _sc_deg_body,
        out_type=jax.ShapeDtypeStruct((2 * _NPAD, _D), jnp.float32),
        mesh=plsc.VectorSubcoreMesh(core_axis_name="c", subcore_axis_name="s"),
        scratch_types=[
            pltpu.VMEM((_DST_RPW, _CH), jnp.int32),
            pltpu.VMEM((_CH, _D), jnp.float32),
            pltpu.VMEM_SHARED((_NPAD, _D), jnp.float32),
            pltpu.SemaphoreType.DMA,
        ],
    )
    return fn(dst3, zeros128, ones128)


def _tc_input_body(f_ref, w_ref, b_ref, degp_ref, h0_ref, g_ref, isd_ref):
    deg = degp_ref[0][:, 0:1] + degp_ref[1][:, 0:1] + 1.0
    isd = lax.rsqrt(deg)
    h = jnp.dot(f_ref[...], w_ref[...], preferred_element_type=jnp.float32)
    h = jnp.maximum(h + b_ref[...], 0.0)
    h0_ref[...] = h
    g_ref[...] = h * isd
    isd_ref[...] = jnp.broadcast_to(isd, (_R, 16))


def _tc_input(features, W_in, b_in, degp):
    return pl.pallas_call(
        _tc_input_body,
        grid=(_GRID,),
        in_specs=[
            pl.BlockSpec((_R, _D), lambda i: (i, 0)),
            pl.BlockSpec((_D, _D), lambda i: (0, 0)),
            pl.BlockSpec((1, _D), lambda i: (0, 0)),
            [pl.BlockSpec((_R, _D), lambda i: (i, 0)),
             pl.BlockSpec((_R, _D), lambda i: (_GRID + i, 0))],
        ],
        out_specs=[
            pl.BlockSpec((_R, _D), lambda i: (i, 0)),
            pl.BlockSpec((_R, _D), lambda i: (i, 0)),
            pl.BlockSpec((_R, 16), lambda i: (i, 0)),
        ],
        out_shape=[
            jax.ShapeDtypeStruct((_NPAD, _D), jnp.float32),
            jax.ShapeDtypeStruct((_NPAD, _D), jnp.float32),
            jax.ShapeDtypeStruct((_NPAD, 16), jnp.float32),
        ],
    )(features, W_in, b_in, [degp, degp])


def _tc_layer_body(beta, p_ref, g_ref, h0_ref, isd_ref, w_ref, gout_ref):
    isd = isd_ref[:, 0:1]
    ssum = p_ref[0][...] + p_ref[1][...] + g_ref[...]
    support = (1.0 - _ALPHA) * (isd * ssum) + _ALPHA * h0_ref[...]
    sw = jnp.dot(support, w_ref[...], preferred_element_type=jnp.float32)
    h = jnp.maximum((1.0 - beta) * support + beta * sw, 0.0)
    gout_ref[...] = h * isd


def _tc_layer(beta, p, g, h0, isd, W):
    return pl.pallas_call(
        functools.partial(_tc_layer_body, beta),
        grid=(_GRID,),
        in_specs=[
            [pl.BlockSpec((_R, _D), lambda i: (i, 0)),
             pl.BlockSpec((_R, _D), lambda i: (_GRID + i, 0))],
            pl.BlockSpec((_R, _D), lambda i: (i, 0)),
            pl.BlockSpec((_R, _D), lambda i: (i, 0)),
            pl.BlockSpec((_R, 16), lambda i: (i, 0)),
            pl.BlockSpec((_D, _D), lambda i: (0, 0)),
        ],
        out_specs=pl.BlockSpec((_R, _D), lambda i: (i, 0)),
        out_shape=jax.ShapeDtypeStruct((_NPAD, _D), jnp.float32),
    )([p, p], g, h0, isd, W)


def _tc_final_body(beta, p_ref, g_ref, h0_ref, isd_ref, w_ref, wout_ref,
                   bout_ref, out_ref):
    isd = isd_ref[:, 0:1]
    ssum = p_ref[0][...] + p_ref[1][...] + g_ref[...]
    support = (1.0 - _ALPHA) * (isd * ssum) + _ALPHA * h0_ref[...]
    sw = jnp.dot(support, w_ref[...], preferred_element_type=jnp.float32)
    h = jnp.maximum((1.0 - beta) * support + beta * sw, 0.0)
    out = jnp.dot(h, wout_ref[...], preferred_element_type=jnp.float32)
    out_ref[...] = out + bout_ref[...]


def _tc_final(beta, p, g, h0, isd, W, W_out, b_out):
    return pl.pallas_call(
        functools.partial(_tc_final_body, beta),
        grid=(_GRID,),
        in_specs=[
            [pl.BlockSpec((_R, _D), lambda i: (i, 0)),
             pl.BlockSpec((_R, _D), lambda i: (_GRID + i, 0))],
            pl.BlockSpec((_R, _D), lambda i: (i, 0)),
            pl.BlockSpec((_R, _D), lambda i: (i, 0)),
            pl.BlockSpec((_R, 16), lambda i: (i, 0)),
            pl.BlockSpec((_D, _D), lambda i: (0, 0)),
            pl.BlockSpec((_D, _D), lambda i: (0, 0)),
            pl.BlockSpec((1, _D), lambda i: (0, 0)),
        ],
        out_specs=pl.BlockSpec((_R, _D), lambda i: (i, 0)),
        out_shape=jax.ShapeDtypeStruct((_N, _D), jnp.float32),
    )([p, p], g, h0, isd, W, W_out, b_out)


def kernel(features, edge_index, W_in, b_in, W_conv, W_out, b_out):
    src = edge_index[0]
    dst = edge_index[1]
    pad = jnp.full((_EPAD - _E,), _N, dtype=jnp.int32)
    srcc = jnp.concatenate([src, pad]).reshape(_NW, _CPW, _CH)
    dstc = jnp.concatenate([dst, pad]).reshape(_NW, _CPW, _CH)
    spad = jnp.full((_NW, _SRCP - _CPP, _CH), _N, dtype=jnp.int32)
    dpad = jnp.full((_NW, _DSTP - _CPP, _CH), _N, dtype=jnp.int32)
    src3 = jnp.concatenate(
        [srcc[:, :_CPP], spad, srcc[:, _CPP:], spad],
        axis=1).reshape(_NW * _SRC_RPW, _CH)
    dst3 = jnp.concatenate(
        [dstc[:, :_CPP], dpad, dstc[:, _CPP:], dpad],
        axis=1).reshape(_NW * _DST_RPW, _CH)
    zeros128 = jnp.zeros((_NPAD, _D), jnp.float32)
    ones128 = jnp.ones((_CH, _D), jnp.float32)
    fpad = jnp.zeros((_NPAD - _N, _D), jnp.float32)
    fpadded = jnp.concatenate([features, fpad], axis=0)

    degp = _sc_degree(dst3, zeros128, ones128)
    h0, g, isd = _tc_input(fpadded, W_in, b_in.reshape(1, _D), degp)
    out = None
    for l in range(_N_LAYERS):
        p = _sc_agg(g, src3, dst3, zeros128)
        beta = math.log(_LAMDA / (l + 1) + 1.0)
        if l < _N_LAYERS - 1:
            g = _tc_layer(beta, p, g, h0, isd, W_conv[l])
        else:
            out = _tc_final(beta, p, g, h0, isd, W_conv[l], W_out,
                            b_out.reshape(1, _D))
    return out


# speedup vs baseline: 2.7339x; 2.7339x over previous
import functools
import math

import jax
import jax.numpy as jnp
from jax import lax
from jax.experimental import pallas as pl
from jax.experimental.pallas import tpu as pltpu
from jax.experimental.pallas import tpu_sc as plsc

_N = 10000
_E = 320000
_D = 128
_N_LAYERS = 4
_ALPHA = 0.1
_LAMDA = 0.5

_NC = 2
_NS = 16
_NW = _NC * _NS

_CH = 128
_CPW = 80
_EPAD = _CH * _CPW * _NW
_CPW0 = 112
_CPW1 = 48
_NPH = 2
_PH = 64
_RPW = _NPH * _PH
_CPP0 = _CPW0 // _NPH
_CPP1 = _CPW1 // _NPH

_NPAD = 10240
_RPT = _NPAD // _NS

_R = 1024
_GRID = _NPAD // _R


def _sc_agg_body(g_hbm, src_hbm, dst_hbm, zeros_hbm, out_hbm,
                 src_v, dst_v, buf, acc, sem):
    c = lax.axis_index("c")
    s = lax.axis_index("s")
    wid = c * _NS + s
    pltpu.sync_copy(zeros_hbm.at[pl.ds(s * _RPT, _RPT)],
                    acc.at[pl.ds(s * _RPT, _RPT)])
    plsc.subcore_barrier()

    nchunks = jnp.where(c == 0, _CPP0, _CPP1)
    for p in range(_NPH):
        pltpu.sync_copy(src_hbm.at[pl.ds(wid * _RPW + p * _PH, _PH)], src_v)
        pltpu.sync_copy(dst_hbm.at[pl.ds(wid * _RPW + p * _PH, _PH)], dst_v)

        def chunk(j, carry):
            pltpu.async_copy(g_hbm.at[src_v.at[j]], buf, sem).wait()
            pltpu.sync_copy(buf, acc.at[dst_v.at[j]], add=True)
            return carry

        lax.fori_loop(0, nchunks, chunk, 0)
    plsc.subcore_barrier()
    pltpu.sync_copy(acc.at[pl.ds(s * _RPT, _RPT)],
                    out_hbm.at[pl.ds(c * _NPAD + s * _RPT, _RPT)])


@jax.jit
def _sc_agg(g, src4, dst4, zeros128):
    fn = pl.kernel(
        _sc_agg_body,
        out_type=jax.ShapeDtypeStruct((2 * _NPAD, _D), jnp.float32),
        mesh=plsc.VectorSubcoreMesh(core_axis_name="c", subcore_axis_name="s"),
        scratch_types=[
            pltpu.VMEM((_PH, _CH), jnp.int32),
            pltpu.VMEM((_PH, _CH), jnp.int32),
            pltpu.VMEM((_CH, _D), jnp.float32),
            pltpu.VMEM_SHARED((_NPAD, _D), jnp.float32),
            pltpu.SemaphoreType.DMA,
        ],
    )
    return fn(g, src4, dst4, zeros128)


def _sc_deg_body(dst_hbm, zeros_hbm, ones_hbm, out_hbm,
                 dst_v, ones_v, acc, sem):
    c = lax.axis_index("c")
    s = lax.axis_index("s")
    wid = s * _NC + c
    pltpu.sync_copy(zeros_hbm.at[pl.ds(s * _RPT, _RPT)],
                    acc.at[pl.ds(s * _RPT, _RPT)])
    pltpu.sync_copy(dst_hbm.at[pl.ds(wid * _CPW, _CPW)], dst_v)
    pltpu.sync_copy(ones_hbm, ones_v)
    plsc.subcore_barrier()

    def chunk(j, carry):
        pltpu.sync_copy(ones_v, acc.at[dst_v.at[j]], add=True)
        return carry

    lax.fori_loop(0, _CPW, chunk, 0)
    plsc.subcore_barrier()
    pltpu.sync_copy(acc.at[pl.ds(s * _RPT, _RPT)],
                    out_hbm.at[pl.ds(c * _NPAD + s * _RPT, _RPT)])


@jax.jit
def _sc_degree(dst3, zeros128, ones128):
    fn = pl.kernel(
        _sc_deg_body,
        out_type=jax.ShapeDtypeStruct((2 * _NPAD, _D), jnp.float32),
        mesh=plsc.VectorSubcoreMesh(core_axis_name="c", subcore_axis_name="s"),
        scratch_types=[
            pltpu.VMEM((_CPW, _CH), jnp.int32),
            pltpu.VMEM((_CH, _D), jnp.float32),
            pltpu.VMEM_SHARED((_NPAD, _D), jnp.float32),
            pltpu.SemaphoreType.DMA,
        ],
    )
    return fn(dst3, zeros128, ones128)


def _tc_input_body(f_ref, w_ref, b_ref, degp_ref, h0_ref, g_ref, isd_ref):
    deg = degp_ref[0][:, 0:1] + degp_ref[1][:, 0:1] + 1.0
    isd = lax.rsqrt(deg)
    h = jnp.dot(f_ref[...], w_ref[...], preferred_element_type=jnp.float32)
    h = jnp.maximum(h + b_ref[...], 0.0)
    h0_ref[...] = h
    g_ref[...] = h * isd
    isd_ref[...] = jnp.broadcast_to(isd, (_R, 16))


def _tc_input(features, W_in, b_in, degp):
    return pl.pallas_call(
        _tc_input_body,
        grid=(_GRID,),
        in_specs=[
            pl.BlockSpec((_R, _D), lambda i: (i, 0)),
            pl.BlockSpec((_D, _D), lambda i: (0, 0)),
            pl.BlockSpec((1, _D), lambda i: (0, 0)),
            [pl.BlockSpec((_R, _D), lambda i: (i, 0)),
             pl.BlockSpec((_R, _D), lambda i: (_GRID + i, 0))],
        ],
        out_specs=[
            pl.BlockSpec((_R, _D), lambda i: (i, 0)),
            pl.BlockSpec((_R, _D), lambda i: (i, 0)),
            pl.BlockSpec((_R, 16), lambda i: (i, 0)),
        ],
        out_shape=[
            jax.ShapeDtypeStruct((_NPAD, _D), jnp.float32),
            jax.ShapeDtypeStruct((_NPAD, _D), jnp.float32),
            jax.ShapeDtypeStruct((_NPAD, 16), jnp.float32),
        ],
    )(features, W_in, b_in, [degp, degp])


def _tc_layer_body(beta, p_ref, g_ref, h0_ref, isd_ref, w_ref, gout_ref):
    isd = isd_ref[:, 0:1]
    ssum = p_ref[0][...] + p_ref[1][...] + g_ref[...]
    support = (1.0 - _ALPHA) * (isd * ssum) + _ALPHA * h0_ref[...]
    sw = jnp.dot(support, w_ref[...], preferred_element_type=jnp.float32)
    h = jnp.maximum((1.0 - beta) * support + beta * sw, 0.0)
    gout_ref[...] = h * isd


def _tc_layer(beta, p, g, h0, isd, W):
    return pl.pallas_call(
        functools.partial(_tc_layer_body, beta),
        grid=(_GRID,),
        in_specs=[
            [pl.BlockSpec((_R, _D), lambda i: (i, 0)),
             pl.BlockSpec((_R, _D), lambda i: (_GRID + i, 0))],
            pl.BlockSpec((_R, _D), lambda i: (i, 0)),
            pl.BlockSpec((_R, _D), lambda i: (i, 0)),
            pl.BlockSpec((_R, 16), lambda i: (i, 0)),
            pl.BlockSpec((_D, _D), lambda i: (0, 0)),
        ],
        out_specs=pl.BlockSpec((_R, _D), lambda i: (i, 0)),
        out_shape=jax.ShapeDtypeStruct((_NPAD, _D), jnp.float32),
    )([p, p], g, h0, isd, W)


def _tc_final_body(beta, p_ref, g_ref, h0_ref, isd_ref, w_ref, wout_ref,
                   bout_ref, out_ref):
    isd = isd_ref[:, 0:1]
    ssum = p_ref[0][...] + p_ref[1][...] + g_ref[...]
    support = (1.0 - _ALPHA) * (isd * ssum) + _ALPHA * h0_ref[...]
    sw = jnp.dot(support, w_ref[...], preferred_element_type=jnp.float32)
    h = jnp.maximum((1.0 - beta) * support + beta * sw, 0.0)
    out = jnp.dot(h, wout_ref[...], preferred_element_type=jnp.float32)
    out_ref[...] = out + bout_ref[...]


def _tc_final(beta, p, g, h0, isd, W, W_out, b_out):
    return pl.pallas_call(
        functools.partial(_tc_final_body, beta),
        grid=(_GRID,),
        in_specs=[
            [pl.BlockSpec((_R, _D), lambda i: (i, 0)),
             pl.BlockSpec((_R, _D), lambda i: (_GRID + i, 0))],
            pl.BlockSpec((_R, _D), lambda i: (i, 0)),
            pl.BlockSpec((_R, _D), lambda i: (i, 0)),
            pl.BlockSpec((_R, 16), lambda i: (i, 0)),
            pl.BlockSpec((_D, _D), lambda i: (0, 0)),
            pl.BlockSpec((_D, _D), lambda i: (0, 0)),
            pl.BlockSpec((1, _D), lambda i: (0, 0)),
        ],
        out_specs=pl.BlockSpec((_R, _D), lambda i: (i, 0)),
        out_shape=jax.ShapeDtypeStruct((_N, _D), jnp.float32),
    )([p, p], g, h0, isd, W, W_out, b_out)


def kernel(features, edge_index, W_in, b_in, W_conv, W_out, b_out):
    src = edge_index[0]
    dst = edge_index[1]
    pad = jnp.full((_EPAD - _E,), _N, dtype=jnp.int32)
    src2 = jnp.concatenate([src, pad]).reshape(_EPAD // _CH, _CH)
    dst2 = jnp.concatenate([dst, pad]).reshape(_EPAD // _CH, _CH)

    def asym_layout(ch2):
        c0 = ch2[:_NS * _CPW0].reshape(_NS, _NPH, _CPP0, _CH)
        c1 = ch2[_NS * _CPW0:].reshape(_NS, _NPH, _CPP1, _CH)
        p0 = jnp.full((_NS, _NPH, _PH - _CPP0, _CH), _N, dtype=jnp.int32)
        p1 = jnp.full((_NS, _NPH, _PH - _CPP1, _CH), _N, dtype=jnp.int32)
        return jnp.concatenate([
            jnp.concatenate([c0, p0], axis=2).reshape(-1, _CH),
            jnp.concatenate([c1, p1], axis=2).reshape(-1, _CH),
        ], axis=0)

    src4 = asym_layout(src2)
    dst4 = asym_layout(dst2)
    zeros128 = jnp.zeros((_NPAD, _D), jnp.float32)
    ones128 = jnp.ones((_CH, _D), jnp.float32)
    fpad = jnp.zeros((_NPAD - _N, _D), jnp.float32)
    fpadded = jnp.concatenate([features, fpad], axis=0)

    degp = _sc_degree(dst2, zeros128, ones128)
    h0, g, isd = _tc_input(fpadded, W_in, b_in.reshape(1, _D), degp)
    out = None
    for l in range(_N_LAYERS):
        p = _sc_agg(g, src4, dst4, zeros128)
        beta = math.log(_LAMDA / (l + 1) + 1.0)
        if l < _N_LAYERS - 1:
            g = _tc_layer(beta, p, g, h0, isd, W_conv[l])
        else:
            out = _tc_final(beta, p, g, h0, isd, W_conv[l], W_out,
                            b_out.reshape(1, _D))
    return out


# speedup vs baseline: 2.8968x; 1.0596x over previous
import functools
import math

import jax
import jax.numpy as jnp
from jax import lax
from jax.experimental import pallas as pl
from jax.experimental.pallas import tpu as pltpu
from jax.experimental.pallas import tpu_sc as plsc

_N = 10000
_E = 320000
_D = 128
_N_LAYERS = 4
_ALPHA = 0.1
_LAMDA = 0.5

_NC = 2
_NS = 16
_NW = _NC * _NS

_CH = 128
_CPW = 80
_EPAD = _CH * _CPW * _NW
_NPH = 2
_PH = _CPW // _NPH
_RPW = _NPH * _PH

_NPAD = 10240
_RPT = _NPAD // _NS

_R = 1024
_GRID = _NPAD // _R


def _sc_agg_body(g_hbm, src_hbm, dst_hbm, zeros_hbm, out_hbm,
                 src_v, dst_v, buf, acc, sem):
    c = lax.axis_index("c")
    s = lax.axis_index("s")
    wid = c * _NS + s
    pltpu.sync_copy(zeros_hbm.at[pl.ds(s * _RPT, _RPT)],
                    acc.at[pl.ds(s * _RPT, _RPT)])
    plsc.subcore_barrier()

    for p in range(_NPH):
        pltpu.sync_copy(src_hbm.at[pl.ds(wid * _RPW + p * _PH, _PH)], src_v)
        pltpu.sync_copy(dst_hbm.at[pl.ds(wid * _RPW + p * _PH, _PH)], dst_v)

        def chunk(j, carry):
            pltpu.async_copy(g_hbm.at[src_v.at[j]], buf, sem).wait()
            pltpu.sync_copy(buf, acc.at[dst_v.at[j]], add=True)
            return carry

        lax.fori_loop(0, _PH, chunk, 0)
    plsc.subcore_barrier()
    pltpu.sync_copy(acc.at[pl.ds(s * _RPT, _RPT)],
                    out_hbm.at[pl.ds(c * _NPAD + s * _RPT, _RPT)])


@jax.jit
def _sc_agg(g, src4, dst4, zeros128):
    fn = pl.kernel(
        _sc_agg_body,
        out_type=jax.ShapeDtypeStruct((2 * _NPAD, _D), jnp.float32),
        mesh=plsc.VectorSubcoreMesh(core_axis_name="c", subcore_axis_name="s"),
        scratch_types=[
            pltpu.VMEM((_PH, _CH), jnp.int32),
            pltpu.VMEM((_PH, _CH), jnp.int32),
            pltpu.VMEM((_CH, _D), jnp.float32),
            pltpu.VMEM_SHARED((_NPAD, _D), jnp.float32),
            pltpu.SemaphoreType.DMA,
        ],
    )
    return fn(g, src4, dst4, zeros128)


def _sc_deg_body(dst_hbm, zeros_hbm, ones_hbm, out_hbm,
                 dst_v, ones_v, acc, sem):
    c = lax.axis_index("c")
    s = lax.axis_index("s")
    wid = s * _NC + c
    pltpu.sync_copy(zeros_hbm.at[pl.ds(s * _RPT, _RPT)],
                    acc.at[pl.ds(s * _RPT, _RPT)])
    pltpu.sync_copy(dst_hbm.at[pl.ds(wid * _CPW, _CPW)], dst_v)
    pltpu.sync_copy(ones_hbm, ones_v)
    plsc.subcore_barrier()

    def chunk(j, carry):
        pltpu.sync_copy(ones_v, acc.at[dst_v.at[j]], add=True)
        return carry

    lax.fori_loop(0, _CPW, chunk, 0)
    plsc.subcore_barrier()
    pltpu.sync_copy(acc.at[pl.ds(s * _RPT, _RPT)],
                    out_hbm.at[pl.ds(c * _NPAD + s * _RPT, _RPT)])


@jax.jit
def _sc_degree(dst3, zeros128, ones128):
    fn = pl.kernel(
        _sc_deg_body,
        out_type=jax.ShapeDtypeStruct((2 * _NPAD, _D), jnp.float32),
        mesh=plsc.VectorSubcoreMesh(core_axis_name="c", subcore_axis_name="s"),
        scratch_types=[
            pltpu.VMEM((_CPW, _CH), jnp.int32),
            pltpu.VMEM((_CH, _D), jnp.float32),
            pltpu.VMEM_SHARED((_NPAD, _D), jnp.float32),
            pltpu.SemaphoreType.DMA,
        ],
    )
    return fn(dst3, zeros128, ones128)


def _tc_h_body(f_ref, w_ref, b_ref, h0_ref):
    h = jnp.dot(f_ref[...], w_ref[...], preferred_element_type=jnp.float32)
    h0_ref[...] = jnp.maximum(h + b_ref[...], 0.0)


def _tc_h(features, W_in, b_in):
    return pl.pallas_call(
        _tc_h_body,
        grid=(_GRID,),
        in_specs=[
            pl.BlockSpec((_R, _D), lambda i: (i, 0)),
            pl.BlockSpec((_D, _D), lambda i: (0, 0)),
            pl.BlockSpec((1, _D), lambda i: (0, 0)),
        ],
        out_specs=pl.BlockSpec((_R, _D), lambda i: (i, 0)),
        out_shape=jax.ShapeDtypeStruct((_NPAD, _D), jnp.float32),
    )(features, W_in, b_in)


def _tc_g_body(h0_ref, degp_ref, g_ref, isd_ref):
    deg = degp_ref[0][:, 0:1] + degp_ref[1][:, 0:1] + 1.0
    isd = lax.rsqrt(deg)
    g_ref[...] = h0_ref[...] * isd
    isd_ref[...] = jnp.broadcast_to(isd, (_R, 16))


def _tc_g(h0, degp):
    return pl.pallas_call(
        _tc_g_body,
        grid=(_GRID,),
        in_specs=[
            pl.BlockSpec((_R, _D), lambda i: (i, 0)),
            [pl.BlockSpec((_R, _D), lambda i: (i, 0)),
             pl.BlockSpec((_R, _D), lambda i: (_GRID + i, 0))],
        ],
        out_specs=[
            pl.BlockSpec((_R, _D), lambda i: (i, 0)),
            pl.BlockSpec((_R, 16), lambda i: (i, 0)),
        ],
        out_shape=[
            jax.ShapeDtypeStruct((_NPAD, _D), jnp.float32),
            jax.ShapeDtypeStruct((_NPAD, 16), jnp.float32),
        ],
    )(h0, [degp, degp])


def _tc_layer_body(beta, p_ref, g_ref, h0_ref, isd_ref, w_ref, gout_ref):
    isd = isd_ref[:, 0:1]
    ssum = p_ref[0][...] + p_ref[1][...] + g_ref[...]
    support = (1.0 - _ALPHA) * (isd * ssum) + _ALPHA * h0_ref[...]
    sw = jnp.dot(support, w_ref[...], preferred_element_type=jnp.float32)
    h = jnp.maximum((1.0 - beta) * support + beta * sw, 0.0)
    gout_ref[...] = h * isd


def _tc_layer(beta, p, g, h0, isd, W):
    return pl.pallas_call(
        functools.partial(_tc_layer_body, beta),
        grid=(_GRID,),
        in_specs=[
            [pl.BlockSpec((_R, _D), lambda i: (i, 0)),
             pl.BlockSpec((_R, _D), lambda i: (_GRID + i, 0))],
            pl.BlockSpec((_R, _D), lambda i: (i, 0)),
            pl.BlockSpec((_R, _D), lambda i: (i, 0)),
            pl.BlockSpec((_R, 16), lambda i: (i, 0)),
            pl.BlockSpec((_D, _D), lambda i: (0, 0)),
        ],
        out_specs=pl.BlockSpec((_R, _D), lambda i: (i, 0)),
        out_shape=jax.ShapeDtypeStruct((_NPAD, _D), jnp.float32),
    )([p, p], g, h0, isd, W)


def _tc_final_body(beta, p_ref, g_ref, h0_ref, isd_ref, w_ref, wout_ref,
                   bout_ref, out_ref):
    isd = isd_ref[:, 0:1]
    ssum = p_ref[0][...] + p_ref[1][...] + g_ref[...]
    support = (1.0 - _ALPHA) * (isd * ssum) + _ALPHA * h0_ref[...]
    sw = jnp.dot(support, w_ref[...], preferred_element_type=jnp.float32)
    h = jnp.maximum((1.0 - beta) * support + beta * sw, 0.0)
    out = jnp.dot(h, wout_ref[...], preferred_element_type=jnp.float32)
    out_ref[...] = out + bout_ref[...]


def _tc_final(beta, p, g, h0, isd, W, W_out, b_out):
    return pl.pallas_call(
        functools.partial(_tc_final_body, beta),
        grid=(_GRID,),
        in_specs=[
            [pl.BlockSpec((_R, _D), lambda i: (i, 0)),
             pl.BlockSpec((_R, _D), lambda i: (_GRID + i, 0))],
            pl.BlockSpec((_R, _D), lambda i: (i, 0)),
            pl.BlockSpec((_R, _D), lambda i: (i, 0)),
            pl.BlockSpec((_R, 16), lambda i: (i, 0)),
            pl.BlockSpec((_D, _D), lambda i: (0, 0)),
            pl.BlockSpec((_D, _D), lambda i: (0, 0)),
            pl.BlockSpec((1, _D), lambda i: (0, 0)),
        ],
        out_specs=pl.BlockSpec((_R, _D), lambda i: (i, 0)),
        out_shape=jax.ShapeDtypeStruct((_N, _D), jnp.float32),
    )([p, p], g, h0, isd, W, W_out, b_out)


def kernel(features, edge_index, W_in, b_in, W_conv, W_out, b_out):
    src = edge_index[0]
    dst = edge_index[1]
    pad = jnp.full((_EPAD - _E,), _N, dtype=jnp.int32)
    src2 = jnp.concatenate([src, pad]).reshape(_EPAD // _CH, _CH)
    dst2 = jnp.concatenate([dst, pad]).reshape(_EPAD // _CH, _CH)
    src4 = src2
    dst4 = dst2
    zeros128 = jnp.zeros((_NPAD, _D), jnp.float32)
    ones128 = jnp.ones((_CH, _D), jnp.float32)
    fpad = jnp.zeros((_NPAD - _N, _D), jnp.float32)
    fpadded = jnp.concatenate([features, fpad], axis=0)

    h0 = _tc_h(fpadded, W_in, b_in.reshape(1, _D))
    degp = _sc_degree(dst2, zeros128, ones128)
    g, isd = _tc_g(h0, degp)
    out = None
    for l in range(_N_LAYERS):
        p = _sc_agg(g, src4, dst4, zeros128)
        beta = math.log(_LAMDA / (l + 1) + 1.0)
        if l < _N_LAYERS - 1:
            g = _tc_layer(beta, p, g, h0, isd, W_conv[l])
        else:
            out = _tc_final(beta, p, g, h0, isd, W_conv[l], W_out,
                            b_out.reshape(1, _D))
    return out


# speedup vs baseline: 3.0181x; 1.0419x over previous
import functools
import math

import jax
import jax.numpy as jnp
from jax import lax
from jax.experimental import pallas as pl
from jax.experimental.pallas import tpu as pltpu
from jax.experimental.pallas import tpu_sc as plsc

_N = 10000
_E = 320000
_D = 128
_N_LAYERS = 4
_ALPHA = 0.1
_LAMDA = 0.5

_NC = 2
_NS = 16
_NW = _NC * _NS

_CH = 128
_CPW = 80
_EPAD = _CH * _CPW * _NW
_NPH = 2
_PH = _CPW // _NPH
_RPW = _NPH * _PH

_NPAD = 10240
_RPT = _NPAD // _NS

_R = 1024
_GRID = _NPAD // _R


def _sc_agg_body(g_hbm, src_hbm, dst_hbm, zeros_hbm, out_hbm,
                 src_v, dst_v, bufa, bufb, acc, gsem, sema, semb):
    c = lax.axis_index("c")
    s = lax.axis_index("s")
    wid = c * _NS + s
    pltpu.sync_copy(zeros_hbm.at[pl.ds(s * _RPT, _RPT)],
                    acc.at[pl.ds(s * _RPT, _RPT)])
    plsc.subcore_barrier()

    for p in range(_NPH):
        pltpu.sync_copy(src_hbm.at[pl.ds(wid * _RPW + p * _PH, _PH)], src_v)
        pltpu.sync_copy(dst_hbm.at[pl.ds(wid * _RPW + p * _PH, _PH)], dst_v)

        def group(i, carry):
            j = 2 * i
            pltpu.async_copy(g_hbm.at[src_v.at[j]], bufa, gsem).wait()
            sa = pltpu.async_copy(bufa, acc.at[dst_v.at[j]], sema, add=True)
            pltpu.async_copy(g_hbm.at[src_v.at[j + 1]], bufb, gsem).wait()
            sa.wait()
            pltpu.async_copy(bufb, acc.at[dst_v.at[j + 1]], semb,
                             add=True).wait()
            return carry

        lax.fori_loop(0, _PH // 2, group, 0)
    plsc.subcore_barrier()
    pltpu.sync_copy(acc.at[pl.ds(s * _RPT, _RPT)],
                    out_hbm.at[pl.ds(c * _NPAD + s * _RPT, _RPT)])


@jax.jit
def _sc_agg(g, src4, dst4, zeros128):
    fn = pl.kernel(
        _sc_agg_body,
        out_type=jax.ShapeDtypeStruct((2 * _NPAD, _D), jnp.float32),
        mesh=plsc.VectorSubcoreMesh(core_axis_name="c", subcore_axis_name="s"),
        scratch_types=[
            pltpu.VMEM((_PH, _CH), jnp.int32),
            pltpu.VMEM((_PH, _CH), jnp.int32),
            pltpu.VMEM((_CH, _D), jnp.float32),
            pltpu.VMEM((_CH, _D), jnp.float32),
            pltpu.VMEM_SHARED((_NPAD, _D), jnp.float32),
            pltpu.SemaphoreType.DMA,
            pltpu.SemaphoreType.DMA,
            pltpu.SemaphoreType.DMA,
        ],
    )
    return fn(g, src4, dst4, zeros128)


def _sc_deg_body(dst_hbm, zeros_hbm, ones_hbm, out_hbm,
                 dst_v, ones_v, acc, sem):
    c = lax.axis_index("c")
    s = lax.axis_index("s")
    wid = s * _NC + c
    pltpu.sync_copy(zeros_hbm.at[pl.ds(s * _RPT, _RPT)],
                    acc.at[pl.ds(s * _RPT, _RPT)])
    pltpu.sync_copy(dst_hbm.at[pl.ds(wid * _CPW, _CPW)], dst_v)
    pltpu.sync_copy(ones_hbm, ones_v)
    plsc.subcore_barrier()

    def chunk(j, carry):
        pltpu.sync_copy(ones_v, acc.at[dst_v.at[j]], add=True)
        return carry

    lax.fori_loop(0, _CPW, chunk, 0)
    plsc.subcore_barrier()
    pltpu.sync_copy(acc.at[pl.ds(s * _RPT, _RPT)],
                    out_hbm.at[pl.ds(c * _NPAD + s * _RPT, _RPT)])


@jax.jit
def _sc_degree(dst3, zeros128, ones128):
    fn = pl.kernel(
        _sc_deg_body,
        out_type=jax.ShapeDtypeStruct((2 * _NPAD, _D), jnp.float32),
        mesh=plsc.VectorSubcoreMesh(core_axis_name="c", subcore_axis_name="s"),
        scratch_types=[
            pltpu.VMEM((_CPW, _CH), jnp.int32),
            pltpu.VMEM((_CH, _D), jnp.float32),
            pltpu.VMEM_SHARED((_NPAD, _D), jnp.float32),
            pltpu.SemaphoreType.DMA,
        ],
    )
    return fn(dst3, zeros128, ones128)


def _tc_h_body(f_ref, w_ref, b_ref, h0_ref):
    h = jnp.dot(f_ref[...], w_ref[...], preferred_element_type=jnp.float32)
    h0_ref[...] = jnp.maximum(h + b_ref[...], 0.0)


def _tc_h(features, W_in, b_in):
    return pl.pallas_call(
        _tc_h_body,
        grid=(_GRID,),
        in_specs=[
            pl.BlockSpec((_R, _D), lambda i: (i, 0)),
            pl.BlockSpec((_D, _D), lambda i: (0, 0)),
            pl.BlockSpec((1, _D), lambda i: (0, 0)),
        ],
        out_specs=pl.BlockSpec((_R, _D), lambda i: (i, 0)),
        out_shape=jax.ShapeDtypeStruct((_NPAD, _D), jnp.float32),
    )(features, W_in, b_in)


def _tc_g_body(h0_ref, degp_ref, g_ref, isd_ref):
    deg = degp_ref[0][:, 0:1] + degp_ref[1][:, 0:1] + 1.0
    isd = lax.rsqrt(deg)
    g_ref[...] = h0_ref[...] * isd
    isd_ref[...] = jnp.broadcast_to(isd, (_R, 16))


def _tc_g(h0, degp):
    return pl.pallas_call(
        _tc_g_body,
        grid=(_GRID,),
        in_specs=[
            pl.BlockSpec((_R, _D), lambda i: (i, 0)),
            [pl.BlockSpec((_R, _D), lambda i: (i, 0)),
             pl.BlockSpec((_R, _D), lambda i: (_GRID + i, 0))],
        ],
        out_specs=[
            pl.BlockSpec((_R, _D), lambda i: (i, 0)),
            pl.BlockSpec((_R, 16), lambda i: (i, 0)),
        ],
        out_shape=[
            jax.ShapeDtypeStruct((_NPAD, _D), jnp.float32),
            jax.ShapeDtypeStruct((_NPAD, 16), jnp.float32),
        ],
    )(h0, [degp, degp])


def _tc_layer_body(beta, p_ref, g_ref, h0_ref, isd_ref, w_ref, gout_ref):
    isd = isd_ref[:, 0:1]
    ssum = p_ref[0][...] + p_ref[1][...] + g_ref[...]
    support = (1.0 - _ALPHA) * (isd * ssum) + _ALPHA * h0_ref[...]
    sw = jnp.dot(support, w_ref[...], preferred_element_type=jnp.float32)
    h = jnp.maximum((1.0 - beta) * support + beta * sw, 0.0)
    gout_ref[...] = h * isd


def _tc_layer(beta, p, g, h0, isd, W):
    return pl.pallas_call(
        functools.partial(_tc_layer_body, beta),
        grid=(_GRID,),
        in_specs=[
            [pl.BlockSpec((_R, _D), lambda i: (i, 0)),
             pl.BlockSpec((_R, _D), lambda i: (_GRID + i, 0))],
            pl.BlockSpec((_R, _D), lambda i: (i, 0)),
            pl.BlockSpec((_R, _D), lambda i: (i, 0)),
            pl.BlockSpec((_R, 16), lambda i: (i, 0)),
            pl.BlockSpec((_D, _D), lambda i: (0, 0)),
        ],
        out_specs=pl.BlockSpec((_R, _D), lambda i: (i, 0)),
        out_shape=jax.ShapeDtypeStruct((_NPAD, _D), jnp.float32),
    )([p, p], g, h0, isd, W)


def _tc_final_body(beta, p_ref, g_ref, h0_ref, isd_ref, w_ref, wout_ref,
                   bout_ref, out_ref):
    isd = isd_ref[:, 0:1]
    ssum = p_ref[0][...] + p_ref[1][...] + g_ref[...]
    support = (1.0 - _ALPHA) * (isd * ssum) + _ALPHA * h0_ref[...]
    sw = jnp.dot(support, w_ref[...], preferred_element_type=jnp.float32)
    h = jnp.maximum((1.0 - beta) * support + beta * sw, 0.0)
    out = jnp.dot(h, wout_ref[...], preferred_element_type=jnp.float32)
    out_ref[...] = out + bout_ref[...]


def _tc_final(beta, p, g, h0, isd, W, W_out, b_out):
    return pl.pallas_call(
        functools.partial(_tc_final_body, beta),
        grid=(_GRID,),
        in_specs=[
            [pl.BlockSpec((_R, _D), lambda i: (i, 0)),
             pl.BlockSpec((_R, _D), lambda i: (_GRID + i, 0))],
            pl.BlockSpec((_R, _D), lambda i: (i, 0)),
            pl.BlockSpec((_R, _D), lambda i: (i, 0)),
            pl.BlockSpec((_R, 16), lambda i: (i, 0)),
            pl.BlockSpec((_D, _D), lambda i: (0, 0)),
            pl.BlockSpec((_D, _D), lambda i: (0, 0)),
            pl.BlockSpec((1, _D), lambda i: (0, 0)),
        ],
        out_specs=pl.BlockSpec((_R, _D), lambda i: (i, 0)),
        out_shape=jax.ShapeDtypeStruct((_N, _D), jnp.float32),
    )([p, p], g, h0, isd, W, W_out, b_out)


def kernel(features, edge_index, W_in, b_in, W_conv, W_out, b_out):
    src = edge_index[0]
    dst = edge_index[1]
    pad = jnp.full((_EPAD - _E,), _N, dtype=jnp.int32)
    src2 = jnp.concatenate([src, pad]).reshape(_EPAD // _CH, _CH)
    dst2 = jnp.concatenate([dst, pad]).reshape(_EPAD // _CH, _CH)
    src4 = src2
    dst4 = dst2
    zeros128 = jnp.zeros((_NPAD, _D), jnp.float32)
    ones128 = jnp.ones((_CH, _D), jnp.float32)
    fpad = jnp.zeros((_NPAD - _N, _D), jnp.float32)
    fpadded = jnp.concatenate([features, fpad], axis=0)

    h0 = _tc_h(fpadded, W_in, b_in.reshape(1, _D))
    degp = _sc_degree(dst2, zeros128, ones128)
    g, isd = _tc_g(h0, degp)
    out = None
    for l in range(_N_LAYERS):
        p = _sc_agg(g, src4, dst4, zeros128)
        beta = math.log(_LAMDA / (l + 1) + 1.0)
        if l < _N_LAYERS - 1:
            g = _tc_layer(beta, p, g, h0, isd, W_conv[l])
        else:
            out = _tc_final(beta, p, g, h0, isd, W_conv[l], W_out,
                            b_out.reshape(1, _D))
    return out


# speedup vs baseline: 3.0240x; 1.0019x over previous
import functools
import math

import jax
import jax.numpy as jnp
from jax import lax
from jax.experimental import pallas as pl
from jax.experimental.pallas import tpu as pltpu
from jax.experimental.pallas import tpu_sc as plsc

_N = 10000
_E = 320000
_D = 128
_N_LAYERS = 4
_ALPHA = 0.1
_LAMDA = 0.5

_NC = 2
_NS = 16
_NW = _NC * _NS

_CH = 128
_CPW = 80
_EPAD = _CH * _CPW * _NW
_NPH = 2
_PH = _CPW // _NPH
_RPW = _NPH * _PH

_NPAD = 10240
_RPT = _NPAD // _NS

_R = 1024
_GRID = _NPAD // _R


def _sc_agg_body(g_hbm, src_hbm, dst_hbm, zeros_hbm, out_hbm,
                 src_v, dst_v, bufa, bufb, acc, gsem, sema, semb):
    c = lax.axis_index("c")
    s = lax.axis_index("s")
    wid = c * _NS + s
    pltpu.sync_copy(zeros_hbm.at[pl.ds(s * _RPT, _RPT)],
                    acc.at[pl.ds(s * _RPT, _RPT)])
    plsc.subcore_barrier()

    for p in range(_NPH):
        pltpu.sync_copy(src_hbm.at[pl.ds(wid * _RPW + p * _PH, _PH)], src_v)
        pltpu.sync_copy(dst_hbm.at[pl.ds(wid * _RPW + p * _PH, _PH)], dst_v)

        def group(i, carry):
            j = 2 * i
            pltpu.async_copy(g_hbm.at[src_v.at[j]], bufa, gsem).wait()
            sa = pltpu.async_copy(bufa, acc.at[dst_v.at[j]], sema, add=True)
            pltpu.async_copy(g_hbm.at[src_v.at[j + 1]], bufb, gsem).wait()
            sa.wait()
            pltpu.async_copy(bufb, acc.at[dst_v.at[j + 1]], semb,
                             add=True).wait()
            return carry

        lax.fori_loop(0, _PH // 2, group, 0)
    plsc.subcore_barrier()
    pltpu.sync_copy(acc.at[pl.ds(s * _RPT, _RPT)],
                    out_hbm.at[pl.ds(c * _NPAD + s * _RPT, _RPT)])


@jax.jit
def _sc_agg(g, src4, dst4, zeros128):
    fn = pl.kernel(
        _sc_agg_body,
        out_type=jax.ShapeDtypeStruct((2 * _NPAD, _D), jnp.float32),
        mesh=plsc.VectorSubcoreMesh(core_axis_name="c", subcore_axis_name="s"),
        scratch_types=[
            pltpu.VMEM((_PH, _CH), jnp.int32),
            pltpu.VMEM((_PH, _CH), jnp.int32),
            pltpu.VMEM((_CH, _D), jnp.float32),
            pltpu.VMEM((_CH, _D), jnp.float32),
            pltpu.VMEM_SHARED((_NPAD, _D), jnp.float32),
            pltpu.SemaphoreType.DMA,
            pltpu.SemaphoreType.DMA,
            pltpu.SemaphoreType.DMA,
        ],
    )
    return fn(g, src4, dst4, zeros128)


def _sc_deg_body(dst_hbm, zeros_hbm, ones_hbm, out_hbm,
                 dst_v, ones_v, acc, sema, semb):
    c = lax.axis_index("c")
    s = lax.axis_index("s")
    wid = s * _NC + c
    pltpu.sync_copy(zeros_hbm.at[pl.ds(s * _RPT, _RPT)],
                    acc.at[pl.ds(s * _RPT, _RPT)])
    pltpu.sync_copy(dst_hbm.at[pl.ds(wid * _CPW, _CPW)], dst_v)
    pltpu.sync_copy(ones_hbm, ones_v)
    plsc.subcore_barrier()

    def group(i, carry):
        j = 2 * i
        sa = pltpu.async_copy(ones_v, acc.at[dst_v.at[j]], sema, add=True)
        sb = pltpu.async_copy(ones_v, acc.at[dst_v.at[j + 1]], semb, add=True)
        sa.wait()
        sb.wait()
        return carry

    lax.fori_loop(0, _CPW // 2, group, 0)
    plsc.subcore_barrier()
    pltpu.sync_copy(acc.at[pl.ds(s * _RPT, _RPT)],
                    out_hbm.at[pl.ds(c * _NPAD + s * _RPT, _RPT)])


@jax.jit
def _sc_degree(dst3, zeros128, ones128):
    fn = pl.kernel(
        _sc_deg_body,
        out_type=jax.ShapeDtypeStruct((2 * _NPAD, _D), jnp.float32),
        mesh=plsc.VectorSubcoreMesh(core_axis_name="c", subcore_axis_name="s"),
        scratch_types=[
            pltpu.VMEM((_CPW, _CH), jnp.int32),
            pltpu.VMEM((_CH, _D), jnp.float32),
            pltpu.VMEM_SHARED((_NPAD, _D), jnp.float32),
            pltpu.SemaphoreType.DMA,
            pltpu.SemaphoreType.DMA,
        ],
    )
    return fn(dst3, zeros128, ones128)


def _tc_h_body(f_ref, w_ref, b_ref, h0_ref):
    h = jnp.dot(f_ref[...], w_ref[...], preferred_element_type=jnp.float32)
    h0_ref[...] = jnp.maximum(h + b_ref[...], 0.0)


def _tc_h(features, W_in, b_in):
    return pl.pallas_call(
        _tc_h_body,
        grid=(_GRID,),
        in_specs=[
            pl.BlockSpec((_R, _D), lambda i: (i, 0)),
            pl.BlockSpec((_D, _D), lambda i: (0, 0)),
            pl.BlockSpec((1, _D), lambda i: (0, 0)),
        ],
        out_specs=pl.BlockSpec((_R, _D), lambda i: (i, 0)),
        out_shape=jax.ShapeDtypeStruct((_NPAD, _D), jnp.float32),
    )(features, W_in, b_in)


def _tc_g_body(h0_ref, degp_ref, g_ref, isd_ref):
    deg = degp_ref[0][:, 0:1] + degp_ref[1][:, 0:1] + 1.0
    isd = lax.rsqrt(deg)
    g_ref[...] = h0_ref[...] * isd
    isd_ref[...] = jnp.broadcast_to(isd, (_R, 16))


def _tc_g(h0, degp):
    return pl.pallas_call(
        _tc_g_body,
        grid=(_GRID,),
        in_specs=[
            pl.BlockSpec((_R, _D), lambda i: (i, 0)),
            [pl.BlockSpec((_R, _D), lambda i: (i, 0)),
             pl.BlockSpec((_R, _D), lambda i: (_GRID + i, 0))],
        ],
        out_specs=[
            pl.BlockSpec((_R, _D), lambda i: (i, 0)),
            pl.BlockSpec((_R, 16), lambda i: (i, 0)),
        ],
        out_shape=[
            jax.ShapeDtypeStruct((_NPAD, _D), jnp.float32),
            jax.ShapeDtypeStruct((_NPAD, 16), jnp.float32),
        ],
    )(h0, [degp, degp])


def _tc_layer_body(beta, p_ref, g_ref, h0_ref, isd_ref, w_ref, gout_ref):
    isd = isd_ref[:, 0:1]
    ssum = p_ref[0][...] + p_ref[1][...] + g_ref[...]
    support = (1.0 - _ALPHA) * (isd * ssum) + _ALPHA * h0_ref[...]
    sw = jnp.dot(support, w_ref[...], preferred_element_type=jnp.float32)
    h = jnp.maximum((1.0 - beta) * support + beta * sw, 0.0)
    gout_ref[...] = h * isd


def _tc_layer(beta, p, g, h0, isd, W):
    return pl.pallas_call(
        functools.partial(_tc_layer_body, beta),
        grid=(_GRID,),
        in_specs=[
            [pl.BlockSpec((_R, _D), lambda i: (i, 0)),
             pl.BlockSpec((_R, _D), lambda i: (_GRID + i, 0))],
            pl.BlockSpec((_R, _D), lambda i: (i, 0)),
            pl.BlockSpec((_R, _D), lambda i: (i, 0)),
            pl.BlockSpec((_R, 16), lambda i: (i, 0)),
            pl.BlockSpec((_D, _D), lambda i: (0, 0)),
        ],
        out_specs=pl.BlockSpec((_R, _D), lambda i: (i, 0)),
        out_shape=jax.ShapeDtypeStruct((_NPAD, _D), jnp.float32),
    )([p, p], g, h0, isd, W)


def _tc_final_body(beta, p_ref, g_ref, h0_ref, isd_ref, w_ref, wout_ref,
                   bout_ref, out_ref):
    isd = isd_ref[:, 0:1]
    ssum = p_ref[0][...] + p_ref[1][...] + g_ref[...]
    support = (1.0 - _ALPHA) * (isd * ssum) + _ALPHA * h0_ref[...]
    sw = jnp.dot(support, w_ref[...], preferred_element_type=jnp.float32)
    h = jnp.maximum((1.0 - beta) * support + beta * sw, 0.0)
    out = jnp.dot(h, wout_ref[...], preferred_element_type=jnp.float32)
    out_ref[...] = out + bout_ref[...]


def _tc_final(beta, p, g, h0, isd, W, W_out, b_out):
    return pl.pallas_call(
        functools.partial(_tc_final_body, beta),
        grid=(_GRID,),
        in_specs=[
            [pl.BlockSpec((_R, _D), lambda i: (i, 0)),
             pl.BlockSpec((_R, _D), lambda i: (_GRID + i, 0))],
            pl.BlockSpec((_R, _D), lambda i: (i, 0)),
            pl.BlockSpec((_R, _D), lambda i: (i, 0)),
            pl.BlockSpec((_R, 16), lambda i: (i, 0)),
            pl.BlockSpec((_D, _D), lambda i: (0, 0)),
            pl.BlockSpec((_D, _D), lambda i: (0, 0)),
            pl.BlockSpec((1, _D), lambda i: (0, 0)),
        ],
        out_specs=pl.BlockSpec((_R, _D), lambda i: (i, 0)),
        out_shape=jax.ShapeDtypeStruct((_N, _D), jnp.float32),
    )([p, p], g, h0, isd, W, W_out, b_out)


def kernel(features, edge_index, W_in, b_in, W_conv, W_out, b_out):
    src = edge_index[0]
    dst = edge_index[1]
    pad = jnp.full((_EPAD - _E,), _N, dtype=jnp.int32)
    src2 = jnp.concatenate([src, pad]).reshape(_EPAD // _CH, _CH)
    dst2 = jnp.concatenate([dst, pad]).reshape(_EPAD // _CH, _CH)
    src4 = src2
    dst4 = dst2
    zeros128 = jnp.zeros((_NPAD, _D), jnp.float32)
    ones128 = jnp.ones((_CH, _D), jnp.float32)
    fpad = jnp.zeros((_NPAD - _N, _D), jnp.float32)
    fpadded = jnp.concatenate([features, fpad], axis=0)

    h0 = _tc_h(fpadded, W_in, b_in.reshape(1, _D))
    degp = _sc_degree(dst2, zeros128, ones128)
    g, isd = _tc_g(h0, degp)
    out = None
    for l in range(_N_LAYERS):
        p = _sc_agg(g, src4, dst4, zeros128)
        beta = math.log(_LAMDA / (l + 1) + 1.0)
        if l < _N_LAYERS - 1:
            g = _tc_layer(beta, p, g, h0, isd, W_conv[l])
        else:
            out = _tc_final(beta, p, g, h0, isd, W_conv[l], W_out,
                            b_out.reshape(1, _D))
    return out
